# baseline (speedup 1.0000x reference)
import functools

import jax
import jax.numpy as jnp
from jax import lax
from jax.experimental import pallas as pl
from jax.experimental.pallas import tpu as pltpu
from jax.experimental.pallas import tpu_sc as plsc

_B, _L, _D = 32, 2048, 1024
_RPW = 2

_mesh = plsc.VectorSubcoreMesh(
    core_axis_name="c", subcore_axis_name="s", num_cores=1
)


@functools.partial(
    pl.kernel,
    mesh=_mesh,
    out_type=jax.ShapeDtypeStruct((_B, _D), jnp.float32),
    scratch_types=[
        pltpu.VMEM((16,), jnp.int32),
        pltpu.VMEM((_RPW, _D), jnp.float32),
        pltpu.VMEM((_RPW, _D), jnp.float32),
        pltpu.SemaphoreType.DMA,
        pltpu.SemaphoreType.DMA,
    ],
)
def _pooler(wt_hbm, mut_hbm, idx_hbm, out_hbm, idx_v, wt_v, mut_v, sem1, sem2):
    w = lax.axis_index("s")
    pltpu.sync_copy(idx_hbm.at[w], idx_v)
    idx2 = idx_v.at[pl.ds(0, _RPW)]
    cp_wt = pltpu.async_copy(wt_hbm.at[idx2], wt_v, sem1)
    cp_mut = pltpu.async_copy(mut_hbm.at[idx2], mut_v, sem2)
    cp_wt.wait()
    cp_mut.wait()
    for j in range(_RPW):
        for k in range(_D // 16):
            sl = pl.ds(k * 16, 16)
            wt_v[j, sl] = wt_v[j, sl] + mut_v[j, sl]
    pltpu.sync_copy(wt_v, out_hbm.at[pl.ds(w * _RPW, _RPW)])


def kernel(wt_embedding, mut_embedding, positions):
    flat = jnp.arange(_B, dtype=jnp.int32) * _L + positions.astype(jnp.int32)
    idx16 = jnp.pad(flat.reshape(_B // _RPW, _RPW), ((0, 0), (0, 16 - _RPW)))
    wt = wt_embedding.reshape(_B * _L, _D)
    mut = mut_embedding.reshape(_B * _L, _D)
    return _pooler(wt, mut, idx16)


# speedup vs baseline: 1.0147x; 1.0147x over previous
import functools

import jax
import jax.numpy as jnp
from jax import lax
from jax.experimental import pallas as pl
from jax.experimental.pallas import tpu as pltpu
from jax.experimental.pallas import tpu_sc as plsc

_B, _L, _D = 32, 2048, 1024
_RPW = 2

_mesh = plsc.VectorSubcoreMesh(
    core_axis_name="c", subcore_axis_name="s", num_cores=1
)


@functools.partial(
    pl.kernel,
    mesh=_mesh,
    out_type=jax.ShapeDtypeStruct((_B, _D), jnp.float32),
    scratch_types=[
        pltpu.VMEM((16,), jnp.int32),
        pltpu.VMEM((_RPW, _D), jnp.float32),
        pltpu.VMEM((_RPW, _D), jnp.float32),
        pltpu.SemaphoreType.DMA,
        pltpu.SemaphoreType.DMA,
    ],
)
def _pooler(wt_hbm, mut_hbm, idx_hbm, out_hbm, idx_v, wt_v, mut_v, sem1, sem2):
    w = lax.axis_index("s")
    pltpu.sync_copy(idx_hbm.at[w], idx_v)
    idx2 = idx_v.at[pl.ds(0, _RPW)]
    cp_wt = pltpu.async_copy(wt_hbm.at[idx2], wt_v, sem1)
    cp_mut = pltpu.async_copy(mut_hbm.at[idx2], mut_v, sem2)
    cp_wt.wait()
    cp_mut.wait()
    outs = []
    for j in range(_RPW):
        for k in range(_D // 16):
            sl = pl.ds(k * 16, 16)
            wt_v[j, sl] = wt_v[j, sl] + mut_v[j, sl]
        outs.append(
            pltpu.async_copy(
                wt_v.at[pl.ds(j, 1)],
                out_hbm.at[pl.ds(w * _RPW + j, 1)],
                sem1 if j == 0 else sem2,
            )
        )
    for cp in outs:
        cp.wait()


def kernel(wt_embedding, mut_embedding, positions):
    flat = jnp.arange(_B, dtype=jnp.int32) * _L + positions.astype(jnp.int32)
    idx16 = jnp.pad(flat.reshape(_B // _RPW, _RPW), ((0, 0), (0, 16 - _RPW)))
    wt = wt_embedding.reshape(_B * _L, _D)
    mut = mut_embedding.reshape(_B * _L, _D)
    return _pooler(wt, mut, idx16)


# speedup vs baseline: 1.0225x; 1.0077x over previous
import functools

import jax
import jax.numpy as jnp
from jax import lax
from jax.experimental import pallas as pl
from jax.experimental.pallas import tpu as pltpu
from jax.experimental.pallas import tpu_sc as plsc

_B, _L, _D = 32, 2048, 1024
_RPW = 2

_mesh = plsc.VectorSubcoreMesh(
    core_axis_name="c", subcore_axis_name="s", num_cores=1
)


@functools.partial(
    pl.kernel,
    mesh=_mesh,
    out_type=jax.ShapeDtypeStruct((_B, _D), jnp.float32),
    scratch_types=[
        pltpu.VMEM((16,), jnp.int32),
        pltpu.VMEM((1, _D), jnp.float32),
        pltpu.VMEM((1, _D), jnp.float32),
        pltpu.VMEM((1, _D), jnp.float32),
        pltpu.VMEM((1, _D), jnp.float32),
        pltpu.SemaphoreType.DMA,
        pltpu.SemaphoreType.DMA,
    ],
)
def _pooler(
    wt_hbm, mut_hbm, idx_hbm, out_hbm,
    idx_v, wt_v0, wt_v1, mut_v0, mut_v1, sem1, sem2,
):
    w = lax.axis_index("s")
    pltpu.sync_copy(idx_hbm.at[w], idx_v)
    sems = (sem1, sem2)
    wt_vs = (wt_v0, wt_v1)
    mut_vs = (mut_v0, mut_v1)
    gathers = []
    for j in range(_RPW):
        idx1 = idx_v.at[pl.ds(8 * j, 1)]
        gathers.append((
            pltpu.async_copy(wt_hbm.at[idx1], wt_vs[j], sems[j]),
            pltpu.async_copy(mut_hbm.at[idx1], mut_vs[j], sems[j]),
        ))
    outs = []
    for j in range(_RPW):
        gathers[j][0].wait()
        gathers[j][1].wait()
        for k in range(_D // 16):
            sl = pl.ds(k * 16, 16)
            wt_vs[j][0, sl] = wt_vs[j][0, sl] + mut_vs[j][0, sl]
        outs.append(
            pltpu.async_copy(
                wt_vs[j],
                out_hbm.at[pl.ds(w * _RPW + j, 1)],
                sems[j],
            )
        )
    for cp in outs:
        cp.wait()


def kernel(wt_embedding, mut_embedding, positions):
    flat = jnp.arange(_B, dtype=jnp.int32) * _L + positions.astype(jnp.int32)
    idx16 = jnp.pad(
        flat.reshape(_B // _RPW, _RPW, 1), ((0, 0), (0, 0), (0, 7))
    ).reshape(_B // _RPW, 16)
    wt = wt_embedding.reshape(_B * _L, _D)
    mut = mut_embedding.reshape(_B * _L, _D)
    return _pooler(wt, mut, idx16)
